# baseline (speedup 1.0000x reference)
import functools

import jax
import jax.numpy as jnp
from jax import lax
from jax.experimental import pallas as pl
from jax.experimental.pallas import tpu as pltpu
from jax.experimental.pallas import tpu_sc as plsc

NB = 15
NC = 32
L = 16
NCORE = 2
NSUB = 16
NW = NCORE * NSUB
N_ROWS = 1000000
CHUNK = 1536
NFULL = N_ROWS // CHUNK
TAIL0 = NFULL * CHUNK
TAILN = N_ROWS - TAIL0
MAXC = 22
NVEC = NC * (CHUNK // L)
TBL = NB * NC
TBL16 = TBL * L
EPAD = 32
PACK = 65536


def _sc_body(probasT_hbm, labels_hbm, elo_hbm, ehi_hbm,
             outpk_hbm, outsp_hbm,
             pb0, pb1, lb0, lb1, elo, ehi, pk, sump, sem0, sem1):
    w = lax.axis_index("s") * NCORE + lax.axis_index("c")
    base = 20 * w + jnp.minimum(w, 11)
    n_my = jnp.where(w < 11, 21, 20)

    pltpu.sync_copy(elo_hbm, elo)
    pltpu.sync_copy(ehi_hbm, ehi)

    zf = jnp.zeros((L,), jnp.float32)
    zi = jnp.zeros((L,), jnp.int32)

    def zero(j, carry):
        pk[pl.ds(j * L, L)] = zi
        sump[pl.ds(j * L, L)] = zf
        return carry

    lax.fori_loop(0, TBL16 // L, zero, 0)

    pbufs = (pb0, pb1)
    lbufs = (lb0, lb1)
    sems = (sem0, sem1)

    def start(slot, b):
        n0 = (base + slot) * CHUNK
        pltpu.async_copy(
            probasT_hbm.at[:, pl.ds(n0, CHUNK)], pbufs[b], sems[b])
        pltpu.async_copy(
            labels_hbm.at[pl.ds(n0, CHUNK)], lbufs[b], sems[b])

    def drain(b):
        pltpu.make_async_copy(
            probasT_hbm.at[:, pl.ds(0, CHUNK)], pbufs[b], sems[b]).wait()
        pltpu.make_async_copy(
            labels_hbm.at[pl.ds(0, CHUNK)], lbufs[b], sems[b]).wait()

    iota = lax.iota(jnp.int32, L)

    def compute(pb, lb):
        @plsc.parallel_loop(0, NVEC // 2, unroll=8)
        def _(v):
            c0 = v & (NC // 2 - 1)
            s = (v >> 4) * L
            lblv = lb[pl.ds(s, L)]
            for sub in range(2):
                c = c0 + sub * (NC // 2)
                csplat = jnp.full((L,), c, jnp.int32)
                cl = jnp.full((L,), c << 4, jnp.int32) | iota
                p = pb[c, pl.ds(s, L)]
                j0 = (p * float(NB)).astype(jnp.int32)
                lo = plsc.load_gather(elo, [j0])
                hi = plsc.load_gather(ehi, [j0])
                valid = jnp.logical_not((p == lo) | (p == hi))
                offs = jnp.where(p > hi, jnp.int32(NC * L),
                                 jnp.where(p < lo, jnp.int32(-NC * L),
                                           jnp.int32(0)))
                idx = j0 * (NC * L) + cl + offs
                packv = jnp.where(lblv == csplat, jnp.int32(PACK + 1),
                                  jnp.int32(PACK))
                plsc.addupdate_scatter(pk, [idx], packv, mask=valid)
                plsc.addupdate_scatter(sump, [idx], p, mask=valid)

    start(0, 0)
    start(1, 1)

    def ring(i, carry):
        for b in range(2):
            chunk = 2 * i + b

            @pl.when(chunk < n_my)
            def _():
                drain(b)
                compute(pbufs[b], lbufs[b])

                @pl.when(chunk + 2 < n_my)
                def _():
                    start(chunk + 2, b)
        return carry

    lax.fori_loop(0, MAXC // 2, ring, 0)

    pltpu.sync_copy(pk, outpk_hbm.at[pl.ds(w * TBL16, TBL16)])
    pltpu.sync_copy(sump, outsp_hbm.at[pl.ds(w * TBL16, TBL16)])


_sc_hist = functools.partial(
    pl.kernel,
    out_type=(jax.ShapeDtypeStruct((NW * TBL16,), jnp.int32),
              jax.ShapeDtypeStruct((NW * TBL16,), jnp.float32)),
    mesh=plsc.VectorSubcoreMesh(core_axis_name="c", subcore_axis_name="s",
                                num_cores=NCORE, num_subcores=NSUB),
    compiler_params=pltpu.CompilerParams(needs_layout_passes=False,
                                         use_tc_tiling_on_sc=True),
    scratch_types=[
        pltpu.VMEM((NC, CHUNK), jnp.float32),
        pltpu.VMEM((NC, CHUNK), jnp.float32),
        pltpu.VMEM((CHUNK,), jnp.int32),
        pltpu.VMEM((CHUNK,), jnp.int32),
        pltpu.VMEM((EPAD,), jnp.float32),
        pltpu.VMEM((EPAD,), jnp.float32),
        pltpu.VMEM((TBL16,), jnp.int32),
        pltpu.VMEM((TBL16,), jnp.float32),
        pltpu.SemaphoreType.DMA,
        pltpu.SemaphoreType.DMA,
    ],
)(_sc_body)


def _tc_final_body(pk_ref, sp_ref, tp_ref, tl_ref, ed_ref, out_ref):
    x4 = pk_ref[...]
    x = jnp.sum(x4, axis=3)
    cnt = jnp.sum((x >> 16).astype(jnp.float32), axis=0)
    ac = jnp.sum((x & 0xFFFF).astype(jnp.float32), axis=0)
    sp = jnp.sum(jnp.sum(sp_ref[...], axis=3), axis=0)

    tp = tp_ref[...]
    tl = tl_ref[...]
    oh = tl == lax.broadcasted_iota(jnp.int32, (NC, TAILN), 0)
    zero = jnp.zeros((), jnp.float32)
    crows, srows, arows = [], [], []
    for b in range(NB):
        lo = ed_ref[b]
        hi = ed_ref[b + 1]
        m = (lo < tp) & (tp < hi)
        crows.append(jnp.sum(jnp.where(m, 1.0, zero), axis=1))
        srows.append(jnp.sum(jnp.where(m, tp, zero), axis=1))
        arows.append(jnp.sum(jnp.where(m & oh, 1.0, zero), axis=1))
    cnt = cnt + jnp.stack(crows)
    sp = sp + jnp.stack(srows)
    ac = ac + jnp.stack(arows)

    tot = jnp.sum(cnt, axis=0, keepdims=True)
    dc = sp - ac
    pos = cnt > 0
    den = jnp.where(pos, cnt * tot, 1.0)
    term = jnp.where(pos, (dc * dc) / den, 0.0)
    out_ref[0, 0] = jnp.sqrt(jnp.sum(term) / float(NC))


_tc_final = pl.pallas_call(
    _tc_final_body,
    out_shape=jax.ShapeDtypeStruct((1, 1), jnp.float32),
    in_specs=[
        pl.BlockSpec(memory_space=pltpu.VMEM),
        pl.BlockSpec(memory_space=pltpu.VMEM),
        pl.BlockSpec(memory_space=pltpu.VMEM),
        pl.BlockSpec(memory_space=pltpu.VMEM),
        pl.BlockSpec(memory_space=pltpu.SMEM),
    ],
    out_specs=pl.BlockSpec(memory_space=pltpu.SMEM),
)


def kernel(probas, labels):
    edges = jnp.linspace(0.0, 1.0, NB + 1, dtype=jnp.float32)
    pad = jnp.full((EPAD - (NB + 2),), 2.0, jnp.float32)
    elo = jnp.concatenate([edges, jnp.float32(2.0)[None], pad])
    ehi = jnp.concatenate([edges[1:], jnp.full((2,), 2.0, jnp.float32), pad])
    probasT = probas.T
    pk, sp = _sc_hist(probasT, labels, elo, ehi)
    tailp = probasT[:, TAIL0:]
    taillab = labels[TAIL0:].reshape(1, TAILN)
    mce = _tc_final(pk.reshape(NW, NB, NC, L), sp.reshape(NW, NB, NC, L),
                    tailp, taillab, edges)
    return mce[0, 0]


# speedup vs baseline: 1.0603x; 1.0603x over previous
import functools

import jax
import jax.numpy as jnp
from jax import lax
from jax.experimental import pallas as pl
from jax.experimental.pallas import tpu as pltpu
from jax.experimental.pallas import tpu_sc as plsc

NB = 15
NC = 32
L = 16
NCORE = 2
NSUB = 16
NW = NCORE * NSUB
N_ROWS = 1000000
CHUNK = 1536
NFULL = N_ROWS // CHUNK
TAIL0 = NFULL * CHUNK
TAILN = N_ROWS - TAIL0
MAXC = 22
NVEC = NC * (CHUNK // L)
TBL = NB * NC
TBL16 = TBL * L
EPAD = 32
PACK = 65536


def _sc_body(probasT_hbm, labels_hbm, elo_hbm, ehi_hbm,
             outpk_hbm, outsp_hbm,
             pb0, pb1, lb0, lb1, elo, ehi, pk, sump, sem0, sem1):
    w = lax.axis_index("s") * NCORE + lax.axis_index("c")
    base = 20 * w + jnp.minimum(w, 11)
    n_my = jnp.where(w < 11, 21, 20)

    pltpu.sync_copy(elo_hbm, elo)
    pltpu.sync_copy(ehi_hbm, ehi)

    zf = jnp.zeros((L,), jnp.float32)
    zi = jnp.zeros((L,), jnp.int32)

    def zero(j, carry):
        pk[pl.ds(j * L, L)] = zi
        sump[pl.ds(j * L, L)] = zf
        return carry

    lax.fori_loop(0, TBL16 // L, zero, 0)

    pbufs = (pb0, pb1)
    lbufs = (lb0, lb1)
    sems = (sem0, sem1)

    def start(slot, b):
        n0 = (base + slot) * CHUNK
        pltpu.async_copy(
            probasT_hbm.at[:, pl.ds(n0, CHUNK)], pbufs[b], sems[b])
        pltpu.async_copy(
            labels_hbm.at[pl.ds(n0, CHUNK)], lbufs[b], sems[b])

    def drain(b):
        pltpu.make_async_copy(
            probasT_hbm.at[:, pl.ds(0, CHUNK)], pbufs[b], sems[b]).wait()
        pltpu.make_async_copy(
            labels_hbm.at[pl.ds(0, CHUNK)], lbufs[b], sems[b]).wait()

    iota = lax.iota(jnp.int32, L)

    def compute(pb, lb):
        @plsc.parallel_loop(0, NVEC // 2, unroll=6)
        def _(v):
            c0 = v & (NC // 2 - 1)
            s = (v >> 4) * L
            lblv = lb[pl.ds(s, L)]
            for sub in range(2):
                c = c0 + sub * (NC // 2)
                csplat = jnp.full((L,), c, jnp.int32)
                cl = jnp.full((L,), c << 4, jnp.int32) | iota
                p = pb[c, pl.ds(s, L)]
                j0 = (p * float(NB)).astype(jnp.int32)
                lo = plsc.load_gather(elo, [j0])
                hi = plsc.load_gather(ehi, [j0])
                valid = jnp.logical_not((p == lo) | (p == hi))
                offs = jnp.where(p > hi, jnp.int32(NC * L),
                                 jnp.where(p < lo, jnp.int32(-NC * L),
                                           jnp.int32(0)))
                idx = j0 * (NC * L) + cl + offs
                packv = jnp.where(lblv == csplat, jnp.int32(PACK + 1),
                                  jnp.int32(PACK))
                plsc.addupdate_scatter(pk, [idx], packv, mask=valid)
                plsc.addupdate_scatter(sump, [idx], p, mask=valid)

    start(0, 0)
    start(1, 1)

    def ring(i, carry):
        for b in range(2):
            chunk = 2 * i + b

            @pl.when(chunk < n_my)
            def _():
                drain(b)
                compute(pbufs[b], lbufs[b])

                @pl.when(chunk + 2 < n_my)
                def _():
                    start(chunk + 2, b)
        return carry

    lax.fori_loop(0, MAXC // 2, ring, 0)

    pltpu.sync_copy(pk, outpk_hbm.at[pl.ds(w * TBL16, TBL16)])
    pltpu.sync_copy(sump, outsp_hbm.at[pl.ds(w * TBL16, TBL16)])


_sc_hist = functools.partial(
    pl.kernel,
    out_type=(jax.ShapeDtypeStruct((NW * TBL16,), jnp.int32),
              jax.ShapeDtypeStruct((NW * TBL16,), jnp.float32)),
    mesh=plsc.VectorSubcoreMesh(core_axis_name="c", subcore_axis_name="s",
                                num_cores=NCORE, num_subcores=NSUB),
    compiler_params=pltpu.CompilerParams(needs_layout_passes=False,
                                         use_tc_tiling_on_sc=True),
    scratch_types=[
        pltpu.VMEM((NC, CHUNK), jnp.float32),
        pltpu.VMEM((NC, CHUNK), jnp.float32),
        pltpu.VMEM((CHUNK,), jnp.int32),
        pltpu.VMEM((CHUNK,), jnp.int32),
        pltpu.VMEM((EPAD,), jnp.float32),
        pltpu.VMEM((EPAD,), jnp.float32),
        pltpu.VMEM((TBL16,), jnp.int32),
        pltpu.VMEM((TBL16,), jnp.float32),
        pltpu.SemaphoreType.DMA,
        pltpu.SemaphoreType.DMA,
    ],
)(_sc_body)


def _tc_final_body(pk_ref, sp_ref, tp_ref, tl_ref, ed_ref, out_ref):
    x4 = pk_ref[...]
    x = jnp.sum(x4, axis=3)
    cnt = jnp.sum((x >> 16).astype(jnp.float32), axis=0)
    ac = jnp.sum((x & 0xFFFF).astype(jnp.float32), axis=0)
    sp = jnp.sum(jnp.sum(sp_ref[...], axis=3), axis=0)

    tp = tp_ref[...]
    tl = tl_ref[...]
    oh = tl == lax.broadcasted_iota(jnp.int32, (NC, TAILN), 0)
    zero = jnp.zeros((), jnp.float32)
    crows, srows, arows = [], [], []
    for b in range(NB):
        lo = ed_ref[b]
        hi = ed_ref[b + 1]
        m = (lo < tp) & (tp < hi)
        crows.append(jnp.sum(jnp.where(m, 1.0, zero), axis=1))
        srows.append(jnp.sum(jnp.where(m, tp, zero), axis=1))
        arows.append(jnp.sum(jnp.where(m & oh, 1.0, zero), axis=1))
    cnt = cnt + jnp.stack(crows)
    sp = sp + jnp.stack(srows)
    ac = ac + jnp.stack(arows)

    tot = jnp.sum(cnt, axis=0, keepdims=True)
    dc = sp - ac
    pos = cnt > 0
    den = jnp.where(pos, cnt * tot, 1.0)
    term = jnp.where(pos, (dc * dc) / den, 0.0)
    out_ref[0, 0] = jnp.sqrt(jnp.sum(term) / float(NC))


_tc_final = pl.pallas_call(
    _tc_final_body,
    out_shape=jax.ShapeDtypeStruct((1, 1), jnp.float32),
    in_specs=[
        pl.BlockSpec(memory_space=pltpu.VMEM),
        pl.BlockSpec(memory_space=pltpu.VMEM),
        pl.BlockSpec(memory_space=pltpu.VMEM),
        pl.BlockSpec(memory_space=pltpu.VMEM),
        pl.BlockSpec(memory_space=pltpu.SMEM),
    ],
    out_specs=pl.BlockSpec(memory_space=pltpu.SMEM),
)


def kernel(probas, labels):
    edges = jnp.linspace(0.0, 1.0, NB + 1, dtype=jnp.float32)
    pad = jnp.full((EPAD - (NB + 2),), 2.0, jnp.float32)
    elo = jnp.concatenate([edges, jnp.float32(2.0)[None], pad])
    ehi = jnp.concatenate([edges[1:], jnp.full((2,), 2.0, jnp.float32), pad])
    probasT = probas.T
    pk, sp = _sc_hist(probasT, labels, elo, ehi)
    tailp = probasT[:, TAIL0:]
    taillab = labels[TAIL0:].reshape(1, TAILN)
    mce = _tc_final(pk.reshape(NW, NB, NC, L), sp.reshape(NW, NB, NC, L),
                    tailp, taillab, edges)
    return mce[0, 0]


# speedup vs baseline: 1.3682x; 1.2904x over previous
import functools

import jax
import jax.numpy as jnp
from jax import lax
from jax.experimental import pallas as pl
from jax.experimental.pallas import tpu as pltpu
from jax.experimental.pallas import tpu_sc as plsc

NB = 15
NC = 32
L = 16
NCORE = 2
NSUB = 16
NW = NCORE * NSUB
N_ROWS = 1000000
CHUNK = 1536
NFULL = N_ROWS // CHUNK
TAIL0 = NFULL * CHUNK
TAILN = N_ROWS - TAIL0
MAXC = 22
NVEC = NC * (CHUNK // L)
TBL = NB * NC
TBL16 = TBL * L
EPAD = 32
PACK = 65536


def _sc_body(probasT_hbm, labels_hbm, elo_hbm, ehi_hbm,
             outpk_hbm, outsp_hbm,
             pb0, pb1, lb0, lb1, elo, ehi, pk, sump, sem0, sem1):
    w = lax.axis_index("s") * NCORE + lax.axis_index("c")
    base = 20 * w + jnp.minimum(w, 11)
    n_my = jnp.where(w < 11, 21, 20)

    pltpu.sync_copy(elo_hbm, elo)
    pltpu.sync_copy(ehi_hbm, ehi)

    zf = jnp.zeros((L,), jnp.float32)
    zi = jnp.zeros((L,), jnp.int32)

    def zero(j, carry):
        pk[pl.ds(j * L, L)] = zi
        sump[pl.ds(j * L, L)] = zf
        return carry

    lax.fori_loop(0, TBL16 // L, zero, 0)

    pbufs = (pb0, pb1)
    lbufs = (lb0, lb1)
    sems = (sem0, sem1)

    def start(slot, b):
        n0 = (base + slot) * CHUNK
        pltpu.async_copy(
            probasT_hbm.at[:, pl.ds(n0, CHUNK)], pbufs[b], sems[b])
        pltpu.async_copy(
            labels_hbm.at[pl.ds(n0, CHUNK)], lbufs[b], sems[b])

    def drain(b):
        pltpu.make_async_copy(
            probasT_hbm.at[:, pl.ds(0, CHUNK)], pbufs[b], sems[b]).wait()
        pltpu.make_async_copy(
            labels_hbm.at[pl.ds(0, CHUNK)], lbufs[b], sems[b]).wait()

    iota = lax.iota(jnp.int32, L)

    def compute(pb, lb):
        @plsc.parallel_loop(0, NVEC // 2, unroll=4)
        def _(v):
            c0 = v & (NC // 2 - 1)
            s = (v >> 4) * L
            lblv = lb[pl.ds(s, L)]
            for sub in range(2):
                c = c0 + sub * (NC // 2)
                csplat = jnp.full((L,), c, jnp.int32)
                cl = jnp.full((L,), c << 4, jnp.int32) | iota
                p = pb[c, pl.ds(s, L)]
                j0 = (p * float(NB)).astype(jnp.int32)
                lo = plsc.load_gather(elo, [j0])
                hi = plsc.load_gather(ehi, [j0])
                valid = jnp.logical_not((p == lo) | (p == hi))
                offs = jnp.where(p > hi, jnp.int32(NC * L),
                                 jnp.where(p < lo, jnp.int32(-NC * L),
                                           jnp.int32(0)))
                idx = j0 * (NC * L) + cl + offs
                packv = jnp.where(lblv == csplat, jnp.int32(PACK + 1),
                                  jnp.int32(PACK))
                plsc.addupdate_scatter(pk, [idx], packv, mask=valid)
                plsc.addupdate_scatter(sump, [idx], p, mask=valid)

    start(0, 0)
    start(1, 1)

    def ring(i, carry):
        for b in range(2):
            chunk = 2 * i + b

            @pl.when(chunk < n_my)
            def _():
                drain(b)
                compute(pbufs[b], lbufs[b])

                @pl.when(chunk + 2 < n_my)
                def _():
                    start(chunk + 2, b)
        return carry

    lax.fori_loop(0, MAXC // 2, ring, 0)

    pltpu.sync_copy(pk, outpk_hbm.at[pl.ds(w * TBL16, TBL16)])
    pltpu.sync_copy(sump, outsp_hbm.at[pl.ds(w * TBL16, TBL16)])


_sc_hist = functools.partial(
    pl.kernel,
    out_type=(jax.ShapeDtypeStruct((NW * TBL16,), jnp.int32),
              jax.ShapeDtypeStruct((NW * TBL16,), jnp.float32)),
    mesh=plsc.VectorSubcoreMesh(core_axis_name="c", subcore_axis_name="s",
                                num_cores=NCORE, num_subcores=NSUB),
    compiler_params=pltpu.CompilerParams(needs_layout_passes=False,
                                         use_tc_tiling_on_sc=True),
    scratch_types=[
        pltpu.VMEM((NC, CHUNK), jnp.float32),
        pltpu.VMEM((NC, CHUNK), jnp.float32),
        pltpu.VMEM((CHUNK,), jnp.int32),
        pltpu.VMEM((CHUNK,), jnp.int32),
        pltpu.VMEM((EPAD,), jnp.float32),
        pltpu.VMEM((EPAD,), jnp.float32),
        pltpu.VMEM((TBL16,), jnp.int32),
        pltpu.VMEM((TBL16,), jnp.float32),
        pltpu.SemaphoreType.DMA,
        pltpu.SemaphoreType.DMA,
    ],
)(_sc_body)


def _tc_final_body(pk_ref, sp_ref, tp_ref, tl_ref, ed_ref, out_ref):
    x4 = pk_ref[...]
    x = jnp.sum(x4, axis=3)
    cnt = jnp.sum((x >> 16).astype(jnp.float32), axis=0)
    ac = jnp.sum((x & 0xFFFF).astype(jnp.float32), axis=0)
    sp = jnp.sum(jnp.sum(sp_ref[...], axis=3), axis=0)

    tp = tp_ref[...]
    tl = tl_ref[...]
    oh = tl == lax.broadcasted_iota(jnp.int32, (NC, TAILN), 0)
    zero = jnp.zeros((), jnp.float32)
    crows, srows, arows = [], [], []
    for b in range(NB):
        lo = ed_ref[b]
        hi = ed_ref[b + 1]
        m = (lo < tp) & (tp < hi)
        crows.append(jnp.sum(jnp.where(m, 1.0, zero), axis=1))
        srows.append(jnp.sum(jnp.where(m, tp, zero), axis=1))
        arows.append(jnp.sum(jnp.where(m & oh, 1.0, zero), axis=1))
    cnt = cnt + jnp.stack(crows)
    sp = sp + jnp.stack(srows)
    ac = ac + jnp.stack(arows)

    tot = jnp.sum(cnt, axis=0, keepdims=True)
    dc = sp - ac
    pos = cnt > 0
    den = jnp.where(pos, cnt * tot, 1.0)
    term = jnp.where(pos, (dc * dc) / den, 0.0)
    out_ref[0, 0] = jnp.sqrt(jnp.sum(term) / float(NC))


_tc_final = pl.pallas_call(
    _tc_final_body,
    out_shape=jax.ShapeDtypeStruct((1, 1), jnp.float32),
    in_specs=[
        pl.BlockSpec(memory_space=pltpu.VMEM),
        pl.BlockSpec(memory_space=pltpu.VMEM),
        pl.BlockSpec(memory_space=pltpu.VMEM),
        pl.BlockSpec(memory_space=pltpu.VMEM),
        pl.BlockSpec(memory_space=pltpu.SMEM),
    ],
    out_specs=pl.BlockSpec(memory_space=pltpu.SMEM),
)


def kernel(probas, labels):
    edges = jnp.linspace(0.0, 1.0, NB + 1, dtype=jnp.float32)
    pad = jnp.full((EPAD - (NB + 2),), 2.0, jnp.float32)
    elo = jnp.concatenate([edges, jnp.float32(2.0)[None], pad])
    ehi = jnp.concatenate([edges[1:], jnp.full((2,), 2.0, jnp.float32), pad])
    probasT = probas.T
    pk, sp = _sc_hist(probasT, labels, elo, ehi)
    tailp = probasT[:, TAIL0:]
    taillab = labels[TAIL0:].reshape(1, TAILN)
    mce = _tc_final(pk.reshape(NW, NB, NC, L), sp.reshape(NW, NB, NC, L),
                    tailp, taillab, edges)
    return mce[0, 0]


# speedup vs baseline: 1.3868x; 1.0136x over previous
import functools

import jax
import jax.numpy as jnp
from jax import lax
from jax.experimental import pallas as pl
from jax.experimental.pallas import tpu as pltpu
from jax.experimental.pallas import tpu_sc as plsc

NB = 15
NC = 32
L = 16
NCORE = 2
NSUB = 16
NW = NCORE * NSUB
N_ROWS = 1000000
CHUNK = 128
NFULL = N_ROWS // CHUNK
TAIL0 = NFULL * CHUNK
TAILN = N_ROWS - TAIL0
MAXC = 246
NVEC = NC * (CHUNK // L)
TBL = NB * NC
TBL16 = TBL * L
EPAD = 32
PACK = 65536


def _sc_body(probasT_hbm, labels_hbm, elo_hbm, ehi_hbm,
             outpk_hbm, outsp_hbm,
             pb0, pb1, lb0, lb1, elo, ehi, pk, sump, sem0, sem1):
    w = lax.axis_index("s") * NCORE + lax.axis_index("c")
    base = 244 * w + jnp.minimum(w, 4)
    n_my = jnp.where(w < 4, 245, 244)

    pltpu.sync_copy(elo_hbm, elo)
    pltpu.sync_copy(ehi_hbm, ehi)

    zf = jnp.zeros((L,), jnp.float32)
    zi = jnp.zeros((L,), jnp.int32)

    def zero(j, carry):
        pk[pl.ds(j * L, L)] = zi
        sump[pl.ds(j * L, L)] = zf
        return carry

    lax.fori_loop(0, TBL16 // L, zero, 0)

    pbufs = (pb0, pb1)
    lbufs = (lb0, lb1)
    sems = (sem0, sem1)

    def start(slot, b):
        n0 = (base + slot) * CHUNK
        pltpu.async_copy(
            probasT_hbm.at[:, pl.ds(n0, CHUNK)], pbufs[b], sems[b])
        pltpu.async_copy(
            labels_hbm.at[pl.ds(n0, CHUNK)], lbufs[b], sems[b])

    def drain(b):
        pltpu.make_async_copy(
            probasT_hbm.at[:, pl.ds(0, CHUNK)], pbufs[b], sems[b]).wait()
        pltpu.make_async_copy(
            labels_hbm.at[pl.ds(0, CHUNK)], lbufs[b], sems[b]).wait()

    iota = lax.iota(jnp.int32, L)

    def compute(pb, lb):
        @plsc.parallel_loop(0, NVEC // 2, unroll=4)
        def _(v):
            c0 = v & (NC // 2 - 1)
            s = (v >> 4) * L
            lblv = lb[pl.ds(s, L)]
            for sub in range(2):
                c = c0 + sub * (NC // 2)
                csplat = jnp.full((L,), c, jnp.int32)
                cl = jnp.full((L,), c << 4, jnp.int32) | iota
                p = pb[c, pl.ds(s, L)]
                j0 = (p * float(NB)).astype(jnp.int32)
                lo = plsc.load_gather(elo, [j0])
                hi = plsc.load_gather(ehi, [j0])
                valid = jnp.logical_not((p == lo) | (p == hi))
                offs = jnp.where(p > hi, jnp.int32(NC * L),
                                 jnp.where(p < lo, jnp.int32(-NC * L),
                                           jnp.int32(0)))
                idx = j0 * (NC * L) + cl + offs
                packv = jnp.where(lblv == csplat, jnp.int32(PACK + 1),
                                  jnp.int32(PACK))
                plsc.addupdate_scatter(pk, [idx], packv, mask=valid)
                plsc.addupdate_scatter(sump, [idx], p, mask=valid)

    start(0, 0)
    start(1, 1)

    def ring(i, carry):
        for b in range(2):
            chunk = 2 * i + b

            @pl.when(chunk < n_my)
            def _():
                drain(b)
                compute(pbufs[b], lbufs[b])

                @pl.when(chunk + 2 < n_my)
                def _():
                    start(chunk + 2, b)
        return carry

    lax.fori_loop(0, MAXC // 2, ring, 0)

    pltpu.sync_copy(pk, outpk_hbm.at[pl.ds(w * TBL16, TBL16)])
    pltpu.sync_copy(sump, outsp_hbm.at[pl.ds(w * TBL16, TBL16)])


_sc_hist = functools.partial(
    pl.kernel,
    out_type=(jax.ShapeDtypeStruct((NW * TBL16,), jnp.int32),
              jax.ShapeDtypeStruct((NW * TBL16,), jnp.float32)),
    mesh=plsc.VectorSubcoreMesh(core_axis_name="c", subcore_axis_name="s",
                                num_cores=NCORE, num_subcores=NSUB),
    compiler_params=pltpu.CompilerParams(needs_layout_passes=False,
                                         use_tc_tiling_on_sc=True),
    scratch_types=[
        pltpu.VMEM((NC, CHUNK), jnp.float32),
        pltpu.VMEM((NC, CHUNK), jnp.float32),
        pltpu.VMEM((CHUNK,), jnp.int32),
        pltpu.VMEM((CHUNK,), jnp.int32),
        pltpu.VMEM((EPAD,), jnp.float32),
        pltpu.VMEM((EPAD,), jnp.float32),
        pltpu.VMEM((TBL16,), jnp.int32),
        pltpu.VMEM((TBL16,), jnp.float32),
        pltpu.SemaphoreType.DMA,
        pltpu.SemaphoreType.DMA,
    ],
)(_sc_body)


def _tc_final_body(pk_ref, sp_ref, tp_ref, tl_ref, ed_ref, out_ref):
    x4 = pk_ref[...]
    x = jnp.sum(x4, axis=3)
    cnt = jnp.sum((x >> 16).astype(jnp.float32), axis=0)
    ac = jnp.sum((x & 0xFFFF).astype(jnp.float32), axis=0)
    sp = jnp.sum(jnp.sum(sp_ref[...], axis=3), axis=0)

    tp = tp_ref[...]
    tl = tl_ref[...]
    oh = tl == lax.broadcasted_iota(jnp.int32, (NC, TAILN), 0)
    zero = jnp.zeros((), jnp.float32)
    crows, srows, arows = [], [], []
    for b in range(NB):
        lo = ed_ref[b]
        hi = ed_ref[b + 1]
        m = (lo < tp) & (tp < hi)
        crows.append(jnp.sum(jnp.where(m, 1.0, zero), axis=1))
        srows.append(jnp.sum(jnp.where(m, tp, zero), axis=1))
        arows.append(jnp.sum(jnp.where(m & oh, 1.0, zero), axis=1))
    cnt = cnt + jnp.stack(crows)
    sp = sp + jnp.stack(srows)
    ac = ac + jnp.stack(arows)

    tot = jnp.sum(cnt, axis=0, keepdims=True)
    dc = sp - ac
    pos = cnt > 0
    den = jnp.where(pos, cnt * tot, 1.0)
    term = jnp.where(pos, (dc * dc) / den, 0.0)
    out_ref[0, 0] = jnp.sqrt(jnp.sum(term) / float(NC))


_tc_final = pl.pallas_call(
    _tc_final_body,
    out_shape=jax.ShapeDtypeStruct((1, 1), jnp.float32),
    in_specs=[
        pl.BlockSpec(memory_space=pltpu.VMEM),
        pl.BlockSpec(memory_space=pltpu.VMEM),
        pl.BlockSpec(memory_space=pltpu.VMEM),
        pl.BlockSpec(memory_space=pltpu.VMEM),
        pl.BlockSpec(memory_space=pltpu.SMEM),
    ],
    out_specs=pl.BlockSpec(memory_space=pltpu.SMEM),
)


def kernel(probas, labels):
    edges = jnp.linspace(0.0, 1.0, NB + 1, dtype=jnp.float32)
    pad = jnp.full((EPAD - (NB + 2),), 2.0, jnp.float32)
    elo = jnp.concatenate([edges, jnp.float32(2.0)[None], pad])
    ehi = jnp.concatenate([edges[1:], jnp.full((2,), 2.0, jnp.float32), pad])
    probasT = probas.T
    pk, sp = _sc_hist(probasT, labels, elo, ehi)
    tailp = probasT[:, TAIL0:]
    taillab = labels[TAIL0:].reshape(1, TAILN)
    mce = _tc_final(pk.reshape(NW, NB, NC, L), sp.reshape(NW, NB, NC, L),
                    tailp, taillab, edges)
    return mce[0, 0]
